# baseline (speedup 1.0000x reference)
import jax
import jax.numpy as jnp
from jax import lax
from jax.experimental import pallas as pl
from jax.experimental.pallas import tpu as pltpu
from jax.experimental.pallas import tpu_sc as plsc

VOCAB = 100000
EMBED = 128
B, T = 16, 2048
ROWS = B * T
NW = 32
TW = T // NW
CB = 2
CROWS = CB * TW
NCH = B // CB
NB = 6
LOOKAHEAD = 3


def _body(idx_hbm, tok_hbm, pos_hbm, out_hbm,
          idx_v, pos_v,
          r0, r1, r2, r3, r4, r5,
          g0, g1, g2, g3, g4, g5,
          s0, s1, s2, s3, s4, s5, psem, isem):
    rows = [r0, r1, r2, r3, r4, r5]
    gsem = [g0, g1, g2, g3, g4, g5]
    ssem = [s0, s1, s2, s3, s4, s5]
    wid = lax.axis_index("s") * 2 + lax.axis_index("c")
    t0 = wid * TW

    c_pos = pltpu.async_copy(pos_hbm.at[pl.ds(t0, TW)], pos_v, psem)
    c_idx = [
        pltpu.async_copy(
            idx_hbm.at[b, pl.ds(t0, TW)],
            idx_v.at[b // CB, pl.ds((b % CB) * TW, TW)], isem)
        for b in range(B)
    ]

    g = {}
    s = {}

    def start_gather(j):
        buf = j % NB
        c_idx[CB * j].wait()
        c_idx[CB * j + 1].wait()
        g[j] = pltpu.async_copy(tok_hbm.at[idx_v.at[j]], rows[buf], gsem[buf])

    for j in range(LOOKAHEAD):
        start_gather(j)

    for j in range(NCH):
        buf = j % NB
        nj = j + LOOKAHEAD
        if nj < NCH:
            pj = nj - NB
            if pj >= 0:
                s[pj][0].wait()
                s[pj][1].wait()
            start_gather(nj)
        g[j].wait()
        if j == 0:
            c_pos.wait()

        stores = []
        for k in range(CB):
            def row_body(r, _, buf=buf, k=k):
                for c in range(EMBED // 16):
                    sl = pl.ds(c * 16, 16)
                    plsc.addupdate(rows[buf].at[k * TW + r, sl], pos_v[r, sl])
                return 0

            lax.fori_loop(0, TW, row_body, 0, unroll=4)
            stores.append(pltpu.async_copy(
                rows[buf].at[pl.ds(k * TW, TW)],
                out_hbm.at[pl.ds((j * CB + k) * T + t0, TW)],
                ssem[buf]))
        s[j] = tuple(stores)

    for j in range(NCH - NB, NCH):
        if j >= 0:
            s[j][0].wait()
            s[j][1].wait()


def kernel(idx, tok_table, pos_table):
    mesh = plsc.VectorSubcoreMesh(core_axis_name="c", subcore_axis_name="s")
    out = pl.kernel(
        _body,
        mesh=mesh,
        out_type=jax.ShapeDtypeStruct((ROWS, EMBED), jnp.float32),
        scratch_types=[
            pltpu.VMEM((NCH, CROWS), jnp.int32),
            pltpu.VMEM((TW, EMBED), jnp.float32),
        ] + [pltpu.VMEM((CROWS, EMBED), jnp.float32)] * NB
          + [pltpu.SemaphoreType.DMA] * (2 * NB + 2),
    )(idx.astype(jnp.int32), tok_table, pos_table)
    return out.reshape(B, T, EMBED)


# speedup vs baseline: 1.0467x; 1.0467x over previous
import jax
import jax.numpy as jnp
from jax import lax
from jax.experimental import pallas as pl
from jax.experimental.pallas import tpu as pltpu
from jax.experimental.pallas import tpu_sc as plsc

VOCAB = 100000
EMBED = 128
B, T = 16, 2048
ROWS = B * T
NW = 32
TW = T // NW
CB = 2
CROWS = CB * TW
NCH = B // CB
NB = 6
LOOKAHEAD = 3


def _body(idx_hbm, tok_hbm, pos_hbm, out_hbm,
          idx_v, pos_v,
          r0, r1, r2, r3, r4, r5,
          g0, g1, g2, g3, g4, g5,
          s0, s1, s2, s3, s4, s5, psem, isem):
    rows = [r0, r1, r2, r3, r4, r5]
    gsem = [g0, g1, g2, g3, g4, g5]
    ssem = [s0, s1, s2, s3, s4, s5]
    wid = lax.axis_index("s") * 2 + lax.axis_index("c")
    t0 = wid * TW

    c_pos = pltpu.async_copy(pos_hbm.at[pl.ds(t0, TW)], pos_v, psem)
    c_idx = [
        pltpu.async_copy(
            idx_hbm.at[b, pl.ds(t0, TW)],
            idx_v.at[b // CB, pl.ds((b % CB) * TW, TW)], isem)
        for b in range(B)
    ]

    g = {}
    s = {}

    def start_gather(j):
        buf = j % NB
        c_idx[CB * j].wait()
        c_idx[CB * j + 1].wait()
        g[j] = pltpu.async_copy(tok_hbm.at[idx_v.at[j]], rows[buf], gsem[buf])

    for j in range(LOOKAHEAD):
        start_gather(j)

    for j in range(NCH):
        buf = j % NB
        nj = j + LOOKAHEAD
        if nj < NCH:
            pj = nj - NB
            if pj >= 0:
                s[pj][0].wait()
                s[pj][1].wait()
            start_gather(nj)
        g[j].wait()
        if j == 0:
            c_pos.wait()

        def row_body(r, _, buf=buf):
            for c in range(EMBED // 16):
                sl = pl.ds(c * 16, 16)
                v = pos_v[r, sl]
                for k in range(CB):
                    plsc.addupdate(rows[buf].at[k * TW + r, sl], v)
            return 0

        lax.fori_loop(0, TW, row_body, 0, unroll=4)

        s[j] = tuple(
            pltpu.async_copy(
                rows[buf].at[pl.ds(k * TW, TW)],
                out_hbm.at[pl.ds((j * CB + k) * T + t0, TW)],
                ssem[buf])
            for k in range(CB))

    for j in range(NCH - NB, NCH):
        if j >= 0:
            s[j][0].wait()
            s[j][1].wait()


def kernel(idx, tok_table, pos_table):
    mesh = plsc.VectorSubcoreMesh(core_axis_name="c", subcore_axis_name="s")
    out = pl.kernel(
        _body,
        mesh=mesh,
        out_type=jax.ShapeDtypeStruct((ROWS, EMBED), jnp.float32),
        scratch_types=[
            pltpu.VMEM((NCH, CROWS), jnp.int32),
            pltpu.VMEM((TW, EMBED), jnp.float32),
        ] + [pltpu.VMEM((CROWS, EMBED), jnp.float32)] * NB
          + [pltpu.SemaphoreType.DMA] * (2 * NB + 2),
    )(idx.astype(jnp.int32), tok_table, pos_table)
    return out.reshape(B, T, EMBED)


# speedup vs baseline: 1.1377x; 1.0870x over previous
import jax
import jax.numpy as jnp
from jax import lax
from jax.experimental import pallas as pl
from jax.experimental.pallas import tpu as pltpu
from jax.experimental.pallas import tpu_sc as plsc

VOCAB = 100000
EMBED = 128
B, T = 16, 2048
ROWS = B * T
NW = 32
TW = T // NW
CB = 2
CROWS = CB * TW
NCH = B // CB
NB = 6
LOOKAHEAD = 4


def _body(idx_hbm, tok_hbm, pos_hbm, out_hbm,
          idx_v, pos_v,
          r0, r1, r2, r3, r4, r5,
          g0, g1, g2, g3, g4, g5,
          s0, s1, s2, s3, s4, s5, psem, isem):
    rows = [r0, r1, r2, r3, r4, r5]
    gsem = [g0, g1, g2, g3, g4, g5]
    ssem = [s0, s1, s2, s3, s4, s5]
    wid = lax.axis_index("s") * 2 + lax.axis_index("c")
    t0 = wid * TW

    c_pos = pltpu.async_copy(pos_hbm.at[pl.ds(t0, TW)], pos_v, psem)
    c_idx = [
        pltpu.async_copy(
            idx_hbm.at[b, pl.ds(t0, TW)],
            idx_v.at[b // CB, pl.ds((b % CB) * TW, TW)], isem)
        for b in range(B)
    ]

    g = {}
    s = {}

    def start_gather(j):
        buf = j % NB
        c_idx[CB * j].wait()
        c_idx[CB * j + 1].wait()
        g[j] = pltpu.async_copy(tok_hbm.at[idx_v.at[j]], rows[buf], gsem[buf])

    for j in range(LOOKAHEAD):
        start_gather(j)

    for j in range(NCH):
        buf = j % NB
        nj = j + LOOKAHEAD
        if nj < NCH:
            pj = nj - NB
            if pj >= 0:
                s[pj][0].wait()
                s[pj][1].wait()
            start_gather(nj)
        g[j].wait()
        if j == 0:
            c_pos.wait()

        def row_body(r, _, buf=buf):
            for c in range(EMBED // 16):
                sl = pl.ds(c * 16, 16)
                v = pos_v[r, sl]
                for k in range(CB):
                    plsc.addupdate(rows[buf].at[k * TW + r, sl], v)
            return 0

        lax.fori_loop(0, TW, row_body, 0, unroll=2)

        s[j] = tuple(
            pltpu.async_copy(
                rows[buf].at[pl.ds(k * TW, TW)],
                out_hbm.at[pl.ds((j * CB + k) * T + t0, TW)],
                ssem[buf])
            for k in range(CB))

    for j in range(NCH - NB, NCH):
        if j >= 0:
            s[j][0].wait()
            s[j][1].wait()


def kernel(idx, tok_table, pos_table):
    mesh = plsc.VectorSubcoreMesh(core_axis_name="c", subcore_axis_name="s")
    out = pl.kernel(
        _body,
        mesh=mesh,
        out_type=jax.ShapeDtypeStruct((ROWS, EMBED), jnp.float32),
        scratch_types=[
            pltpu.VMEM((NCH, CROWS), jnp.int32),
            pltpu.VMEM((TW, EMBED), jnp.float32),
        ] + [pltpu.VMEM((CROWS, EMBED), jnp.float32)] * NB
          + [pltpu.SemaphoreType.DMA] * (2 * NB + 2),
    )(idx.astype(jnp.int32), tok_table, pos_table)
    return out.reshape(B, T, EMBED)
